# baseline (speedup 1.0000x reference)
import functools

import jax
import jax.numpy as jnp
from jax import lax
from jax.experimental import pallas as pl
from jax.experimental.pallas import tpu as pltpu
from jax.experimental.pallas import tpu_sc as plsc

D = 256
K = 512
BLK = 1024
PAD = -1
CW = 0.25


def _tc_body(flat_ref, mask_ref, embed_ref, iota_ref, idxg_ref, idxo_ref,
             loss_ref):
    i = pl.program_id(0)
    f = flat_ref[...]
    e = embed_ref[...]
    fg = lax.dot_general(f, e, (((1,), (1,)), ((), ())),
                         preferred_element_type=jnp.float32)
    f2 = jnp.sum(f * f, axis=1, keepdims=True)
    e2 = jnp.sum(e * e, axis=1)
    dist = f2 - 2.0 * fg + e2[None, :]
    md = jnp.min(dist, axis=1, keepdims=True)
    idxf = jnp.min(jnp.where(dist <= md, iota_ref[...], float(K)), axis=1)
    idx2 = idxf.astype(jnp.int32).reshape(BLK // 128, 128)
    m = mask_ref[...] > 0
    idxg_ref[...] = jnp.where(m, idx2, K)
    idxo_ref[...] = jnp.where(m, idx2, PAD)
    s = jnp.sum(md)

    @pl.when(i == 0)
    def _():
        loss_ref[...] = jnp.zeros_like(loss_ref)

    loss_ref[...] += s


def _tc_stage(flat, mask2d, embed):
    rows = flat.shape[0]
    nblk = rows // BLK
    sub = BLK // 128
    iota = jnp.arange(K, dtype=jnp.float32).reshape(1, K)
    return pl.pallas_call(
        _tc_body,
        grid=(nblk,),
        in_specs=[
            pl.BlockSpec((BLK, D), lambda i: (i, 0)),
            pl.BlockSpec((sub, 128), lambda i: (i, 0)),
            pl.BlockSpec((K, D), lambda i: (0, 0)),
            pl.BlockSpec((1, K), lambda i: (0, 0)),
        ],
        out_specs=[
            pl.BlockSpec((sub, 128), lambda i: (i, 0)),
            pl.BlockSpec((sub, 128), lambda i: (i, 0)),
            pl.BlockSpec((1, 1), lambda i: (0, 0)),
        ],
        out_shape=[
            jax.ShapeDtypeStruct((rows // 128, 128), jnp.int32),
            jax.ShapeDtypeStruct((rows // 128, 128), jnp.int32),
            jax.ShapeDtypeStruct((1, 1), jnp.float32),
        ],
    )(flat, mask2d, embed, iota)


def _sc_gather(table3, idx2, rows):
    info = plsc.get_sparse_core_info()
    ns = info.num_subcores
    half = D // 2
    per_s = rows // ns
    mesh = plsc.VectorSubcoreMesh(core_axis_name="c", subcore_axis_name="s")

    @functools.partial(
        pl.kernel,
        mesh=mesh,
        out_type=jax.ShapeDtypeStruct((rows, D), jnp.float32),
        scratch_types=[
            pltpu.VMEM(((K + 1) * half,), jnp.float32),
            pltpu.VMEM((per_s,), jnp.int32),
            pltpu.SemaphoreType.DMA,
        ],
    )
    def k(table_hbm, idx_hbm, out_hbm, tab_v, idx_v, sem0):
        c = lax.axis_index("c")
        s = lax.axis_index("s")
        pltpu.sync_copy(table_hbm.at[c], tab_v)
        pltpu.sync_copy(idx_hbm.at[s], idx_v)
        base = s * per_s

        def g_body(g, carry):
            idxv = idx_v[pl.ds(g * 16, 16)]
            r0 = base + g * 16
            for l in range(16):
                off = idxv[l] * half
                pltpu.async_copy(
                    tab_v.at[pl.ds(off, half)],
                    out_hbm.at[r0 + l, pl.ds(c * half, half)],
                    sem0)
            return carry

        lax.fori_loop(0, per_s // 16, g_body, 0)

        def w_body(g, carry):
            for l in range(16):
                pltpu.make_async_copy(
                    tab_v.at[pl.ds(0, half)],
                    out_hbm.at[base + l, pl.ds(c * half, half)],
                    sem0).wait()
            return carry

        lax.fori_loop(0, per_s // 16, w_body, 0)

    return k(table3, idx2)


def kernel(z, phoneme_mask, embed):
    B, N, Dz = z.shape
    rows = B * N
    half_rows = rows // 2
    flat = z.reshape(rows, Dz)
    mask2d = phoneme_mask.reshape(rows // 128, 128).astype(jnp.int32)
    table = jnp.concatenate([embed, jnp.zeros((1, Dz), jnp.float32)], axis=0)
    table3 = table.reshape(K + 1, 2, Dz // 2).transpose(1, 0, 2).reshape(2, -1)
    mh = mask2d.shape[0] // 2
    idxg_a, idxo_a, loss_a = _tc_stage(flat[:half_rows], mask2d[:mh], embed)
    quant_a = _sc_gather(table3, idxg_a.reshape(16, half_rows // 16), half_rows)
    idxg_b, idxo_b, loss_b = _tc_stage(flat[half_rows:], mask2d[mh:], embed)
    quant_b = _sc_gather(table3, idxg_b.reshape(16, half_rows // 16), half_rows)
    quantize = jnp.concatenate([quant_a, quant_b], axis=0).reshape(B, N, Dz)
    indices = jnp.concatenate([idxo_a, idxo_b], axis=0).reshape(B, N)
    commit_loss = (loss_a[0, 0] + loss_b[0, 0]) * (CW / (rows * Dz))
    return (quantize, indices, commit_loss)


# speedup vs baseline: 1.6272x; 1.6272x over previous
import functools

import jax
import jax.numpy as jnp
from jax import lax
from jax.experimental import pallas as pl
from jax.experimental.pallas import tpu as pltpu
from jax.experimental.pallas import tpu_sc as plsc

D = 256
K = 512
BLK = 1024
PAD = -1
CW = 0.25


def _tc_body(flat_ref, mask_ref, embed_ref, iota_ref, idxg_ref, idxo_ref,
             loss_ref):
    i = pl.program_id(0)
    f = flat_ref[...]
    e = embed_ref[...]
    fg = lax.dot_general(f, e, (((1,), (1,)), ((), ())),
                         preferred_element_type=jnp.float32)
    f2 = jnp.sum(f * f, axis=1, keepdims=True)
    e2 = jnp.sum(e * e, axis=1)
    dist = f2 - 2.0 * fg + e2[None, :]
    md = jnp.min(dist, axis=1, keepdims=True)
    idxf = jnp.min(jnp.where(dist <= md, iota_ref[...], float(K)), axis=1)
    idx2 = idxf.astype(jnp.int32).reshape(BLK // 128, 128)
    m = mask_ref[...] > 0
    idxg_ref[...] = jnp.where(m, idx2, K)
    idxo_ref[...] = jnp.where(m, idx2, PAD)
    s = jnp.sum(md)

    @pl.when(i == 0)
    def _():
        loss_ref[...] = jnp.zeros_like(loss_ref)

    loss_ref[...] += s


def _tc_stage(flat, mask2d, embed):
    rows = flat.shape[0]
    nblk = rows // BLK
    sub = BLK // 128
    iota = jnp.arange(K, dtype=jnp.float32).reshape(1, K)
    return pl.pallas_call(
        _tc_body,
        grid=(nblk,),
        in_specs=[
            pl.BlockSpec((BLK, D), lambda i: (i, 0)),
            pl.BlockSpec((sub, 128), lambda i: (i, 0)),
            pl.BlockSpec((K, D), lambda i: (0, 0)),
            pl.BlockSpec((1, K), lambda i: (0, 0)),
        ],
        out_specs=[
            pl.BlockSpec((sub, 128), lambda i: (i, 0)),
            pl.BlockSpec((sub, 128), lambda i: (i, 0)),
            pl.BlockSpec((1, 1), lambda i: (0, 0)),
        ],
        out_shape=[
            jax.ShapeDtypeStruct((rows // 128, 128), jnp.int32),
            jax.ShapeDtypeStruct((rows // 128, 128), jnp.int32),
            jax.ShapeDtypeStruct((1, 1), jnp.float32),
        ],
    )(flat, mask2d, embed, iota)


def _sc_gather(table3, idx2, rows):
    info = plsc.get_sparse_core_info()
    ns = info.num_subcores
    half = D // 2
    per_s = rows // ns
    mesh = plsc.VectorSubcoreMesh(core_axis_name="c", subcore_axis_name="s")

    @functools.partial(
        pl.kernel,
        mesh=mesh,
        out_type=jax.ShapeDtypeStruct((rows, D), jnp.float32),
        scratch_types=[
            pltpu.VMEM(((K + 1) * half,), jnp.float32),
            pltpu.VMEM((per_s,), jnp.int32),
            pltpu.SemaphoreType.DMA,
        ],
    )
    def k(table_hbm, idx_hbm, out_hbm, tab_v, idx_v, sem0):
        c = lax.axis_index("c")
        s = lax.axis_index("s")
        pltpu.sync_copy(table_hbm.at[c], tab_v)
        pltpu.sync_copy(idx_hbm.at[s], idx_v)
        base = s * per_s

        def g_body(g, carry):
            idxv = idx_v[pl.ds(g * 16, 16)]
            r0 = base + g * 16
            for l in range(16):
                off = idxv[l] * half
                pltpu.async_copy(
                    tab_v.at[pl.ds(off, half)],
                    out_hbm.at[r0 + l, pl.ds(c * half, half)],
                    sem0)
            return carry

        lax.fori_loop(0, per_s // 16, g_body, 0)

        def w_body(g, carry):
            for l in range(16):
                pltpu.make_async_copy(
                    tab_v.at[pl.ds(0, half)],
                    out_hbm.at[base + l, pl.ds(c * half, half)],
                    sem0).wait()
            return carry

        lax.fori_loop(0, per_s // 16, w_body, 0)

    return k(table3, idx2)


def kernel(z, phoneme_mask, embed):
    B, N, Dz = z.shape
    rows = B * N
    flat = z.reshape(rows, Dz)
    mask2d = phoneme_mask.reshape(rows // 128, 128).astype(jnp.int32)
    idxg, idxo, loss = _tc_stage(flat, mask2d, embed)
    table = jnp.concatenate([embed, jnp.zeros((1, Dz), jnp.float32)], axis=0)
    table3 = table.reshape(K + 1, 2, Dz // 2).transpose(1, 0, 2).reshape(2, -1)
    idx2 = idxg.reshape(16, rows // 16)
    quant = _sc_gather(table3, idx2, rows)
    quantize = quant.reshape(B, N, Dz)
    indices = idxo.reshape(B, N)
    commit_loss = loss[0, 0] * (CW / (rows * Dz))
    return (quantize, indices, commit_loss)
